# Initial kernel scaffold; baseline (speedup 1.0000x reference)
import jax
import jax.numpy as jnp
from jax.experimental import pallas as pl


def kernel(edge_attr, emb_table):
    raise NotImplementedError("write your pallas kernel here")


# speedup vs baseline: 1.9287x; 1.9287x over previous
import functools

import jax
import jax.numpy as jnp
from jax import lax
from jax.experimental import pallas as pl
from jax.experimental.pallas import tpu as pltpu
from jax.experimental.pallas import tpu_sc as plsc

NUM_RELATIONS = 500
DIM_EDGE = 256
N_EDGES = 160000

NC = 2
NS = 16
NW = NC * NS
BPW = N_EDGES // NW
C = 40
NCHUNK = BPW // C


def _body(idx_hbm, table_hbm, out_hbm, idx_v, rows_v, gsem):
    wid = lax.axis_index("s") * NC + lax.axis_index("c")
    base = wid * BPW
    pltpu.sync_copy(idx_hbm.at[wid], idx_v)

    def chunk(j, carry):
        pltpu.async_copy(table_hbm.at[idx_v.at[j]], rows_v, gsem).wait()
        pltpu.sync_copy(rows_v, out_hbm.at[pl.ds(base + j * C, C)])
        return carry

    lax.fori_loop(0, NCHUNK, chunk, 0)


@functools.partial(
    pl.kernel,
    out_type=jax.ShapeDtypeStruct((N_EDGES, DIM_EDGE), jnp.float32),
    mesh=plsc.VectorSubcoreMesh(core_axis_name="c", subcore_axis_name="s"),
    scratch_types=[
        pltpu.VMEM((NCHUNK, C), jnp.int32),
        pltpu.VMEM((C, DIM_EDGE), jnp.float32),
        pltpu.SemaphoreType.DMA,
    ],
)
def _gather_kernel(idx_hbm, table_hbm, out_hbm, idx_v, rows_v, gsem):
    _body(idx_hbm, table_hbm, out_hbm, idx_v, rows_v, gsem)


def kernel(edge_attr, emb_table):
    idx = edge_attr.astype(jnp.int32).reshape(NW, NCHUNK, C)
    return _gather_kernel(idx, emb_table)


# speedup vs baseline: 2.5736x; 1.3344x over previous
import functools

import jax
import jax.numpy as jnp
from jax import lax
from jax.experimental import pallas as pl
from jax.experimental.pallas import tpu as pltpu
from jax.experimental.pallas import tpu_sc as plsc

NUM_RELATIONS = 500
DIM_EDGE = 256
N_EDGES = 160000

NC = 2
NS = 16
NW = NC * NS
BPW = N_EDGES // NW
C = 40
NCHUNK = BPW // C
NBUF = 5
NGROUP = NCHUNK // NBUF


def _body(idx_hbm, table_hbm, out_hbm, idx_v, rows_v, gsems, ssems):
    wid = lax.axis_index("s") * NC + lax.axis_index("c")
    base = wid * BPW
    pltpu.sync_copy(idx_hbm.at[wid], idx_v)

    def gstart(b, j):
        pltpu.make_async_copy(
            table_hbm.at[idx_v.at[j]], rows_v.at[b], gsems[b]
        ).start()

    def gwait(b):
        pltpu.make_async_copy(
            table_hbm.at[idx_v.at[0]], rows_v.at[b], gsems[b]
        ).wait()

    def sstart(b, j):
        pltpu.make_async_copy(
            rows_v.at[b], out_hbm.at[pl.ds(base + j * C, C)], ssems[b]
        ).start()

    def swait(b):
        pltpu.make_async_copy(
            rows_v.at[b], out_hbm.at[pl.ds(base, C)], ssems[b]
        ).wait()

    for b in range(NBUF):
        gstart(b, b)

    def group(g, carry):
        for b in range(NBUF):
            gwait(b)
            sstart(b, g * NBUF + b)

        @pl.when(g < NGROUP - 1)
        def _():
            for b in range(NBUF):
                swait(b)
                gstart(b, (g + 1) * NBUF + b)

        return carry

    lax.fori_loop(0, NGROUP, group, 0)

    for b in range(NBUF):
        swait(b)


@functools.partial(
    pl.kernel,
    out_type=jax.ShapeDtypeStruct((N_EDGES, DIM_EDGE), jnp.float32),
    mesh=plsc.VectorSubcoreMesh(core_axis_name="c", subcore_axis_name="s"),
    scratch_types=[
        pltpu.VMEM((NCHUNK, C), jnp.int32),
        pltpu.VMEM((NBUF, C, DIM_EDGE), jnp.float32),
    ]
    + [pltpu.SemaphoreType.DMA] * (2 * NBUF),
)
def _gather_kernel(idx_hbm, table_hbm, out_hbm, idx_v, rows_v, *sems):
    _body(idx_hbm, table_hbm, out_hbm, idx_v, rows_v, sems[:NBUF], sems[NBUF:])


def kernel(edge_attr, emb_table):
    idx = edge_attr.astype(jnp.int32).reshape(NW, NCHUNK, C)
    return _gather_kernel(idx, emb_table)
